# baseline (speedup 1.0000x reference)
import jax
import jax.numpy as jnp
from jax.experimental import pallas as pl

TOPK = 8


def _router_block(x_ref, w_ref, b_ref, wts_ref, idx_ref):
    x = x_ref[...]
    w = w_ref[...]
    logits = jax.lax.dot_general(
        x.astype(jnp.bfloat16), w.astype(jnp.bfloat16),
        (((1,), (1,)), ((), ())),
        preferred_element_type=jnp.float32,
    )
    logits = logits + b_ref[...]
    bt, e = logits.shape
    eidx = jax.lax.broadcasted_iota(jnp.int32, (bt, e), 1)
    vals = logits
    vs, idxs = [], []
    for _ in range(TOPK):
        m = jnp.max(vals, axis=1, keepdims=True)
        am = jnp.min(jnp.where(vals == m, eidx, e), axis=1, keepdims=True)
        vs.append(m)
        idxs.append(am)
        vals = jnp.where(eidx == am, -jnp.inf, vals)
    v = jnp.concatenate(vs, axis=1)
    i = jnp.concatenate(idxs, axis=1)
    ex = jnp.exp(v - v[:, :1])
    wts_ref[...] = ex / jnp.sum(ex, axis=1, keepdims=True)
    idx_ref[...] = i


def kernel(x, W, b):
    B, S, H = x.shape
    E = W.shape[0]
    T = B * S
    BT = 256
    xf = x.reshape(T, H)
    wts, idx = pl.pallas_call(
        _router_block,
        grid=(T // BT,),
        in_specs=[
            pl.BlockSpec((BT, H), lambda i: (i, 0)),
            pl.BlockSpec((E, H), lambda i: (0, 0)),
            pl.BlockSpec((1, E), lambda i: (0, 0)),
        ],
        out_specs=[
            pl.BlockSpec((BT, TOPK), lambda i: (i, 0)),
            pl.BlockSpec((BT, TOPK), lambda i: (i, 0)),
        ],
        out_shape=[
            jax.ShapeDtypeStruct((T, TOPK), jnp.float32),
            jax.ShapeDtypeStruct((T, TOPK), jnp.int32),
        ],
    )(xf, W, b.reshape(1, E))
    return wts.reshape(B, S, TOPK), idx.reshape(B, S, TOPK)


# speedup vs baseline: 1.5050x; 1.5050x over previous
import jax
import jax.numpy as jnp
from jax.experimental import pallas as pl

TOPK = 8


def _router_block(x_ref, w_ref, b_ref, wts_ref, idx_ref):
    x = x_ref[...]
    w = w_ref[...]
    logits = jax.lax.dot_general(
        x.astype(jnp.bfloat16), w.astype(jnp.bfloat16),
        (((1,), (1,)), ((), ())),
        preferred_element_type=jnp.float32,
    )
    logits = logits + b_ref[...]
    bt, e = logits.shape
    eidx = jax.lax.broadcasted_iota(jnp.int32, (bt, e), 1)
    vals = logits
    vs, idxs = [], []
    for _ in range(TOPK):
        m = jnp.max(vals, axis=1, keepdims=True)
        am = jnp.min(jnp.where(vals == m, eidx, e), axis=1, keepdims=True)
        vs.append(m)
        idxs.append(am)
        vals = jnp.where(eidx == am, -jnp.inf, vals)
    v = jnp.concatenate(vs, axis=1)
    i = jnp.concatenate(idxs, axis=1)
    ex = jnp.exp(v - v[:, :1])
    wts_ref[...] = ex / jnp.sum(ex, axis=1, keepdims=True)
    idx_ref[...] = i


def kernel(x, W, b):
    B, S, H = x.shape
    E = W.shape[0]
    T = B * S
    BT = 1024
    xf = x.reshape(T, H)
    wts, idx = pl.pallas_call(
        _router_block,
        grid=(T // BT,),
        in_specs=[
            pl.BlockSpec((BT, H), lambda i: (i, 0)),
            pl.BlockSpec((E, H), lambda i: (0, 0)),
            pl.BlockSpec((1, E), lambda i: (0, 0)),
        ],
        out_specs=[
            pl.BlockSpec((BT, TOPK), lambda i: (i, 0)),
            pl.BlockSpec((BT, TOPK), lambda i: (i, 0)),
        ],
        out_shape=[
            jax.ShapeDtypeStruct((T, TOPK), jnp.float32),
            jax.ShapeDtypeStruct((T, TOPK), jnp.int32),
        ],
    )(xf, W, b.reshape(1, E))
    return wts.reshape(B, S, TOPK), idx.reshape(B, S, TOPK)


# speedup vs baseline: 1.5056x; 1.0005x over previous
import jax
import jax.numpy as jnp
from jax.experimental import pallas as pl
from jax.experimental.pallas import tpu as pltpu

TOPK = 8


def _router_block(x_ref, w_ref, b_ref, wts_ref, idx_ref):
    x = x_ref[...]
    w = w_ref[...]
    logits = jax.lax.dot_general(
        x.astype(jnp.bfloat16), w.astype(jnp.bfloat16),
        (((1,), (1,)), ((), ())),
        preferred_element_type=jnp.float32,
    )
    logits = logits + b_ref[...]
    bt, e = logits.shape
    eidx = jax.lax.broadcasted_iota(jnp.int32, (bt, e), 1)
    vals = logits
    vs, idxs = [], []
    for _ in range(TOPK):
        m = jnp.max(vals, axis=1, keepdims=True)
        am = jnp.min(jnp.where(vals == m, eidx, e), axis=1, keepdims=True)
        vs.append(m)
        idxs.append(am)
        vals = jnp.where(eidx == am, -jnp.inf, vals)
    v = jnp.concatenate(vs, axis=1)
    i = jnp.concatenate(idxs, axis=1)
    ex = jnp.exp(v - v[:, :1])
    wts_ref[...] = ex / jnp.sum(ex, axis=1, keepdims=True)
    idx_ref[...] = i


def kernel(x, W, b):
    B, S, H = x.shape
    E = W.shape[0]
    T = B * S
    BT = 1024
    xf = x.reshape(T, H)
    wts, idx = pl.pallas_call(
        _router_block,
        grid=(T // BT,),
        in_specs=[
            pl.BlockSpec((BT, H), lambda i: (i, 0)),
            pl.BlockSpec((E, H), lambda i: (0, 0)),
            pl.BlockSpec((1, E), lambda i: (0, 0)),
        ],
        out_specs=[
            pl.BlockSpec((BT, TOPK), lambda i: (i, 0)),
            pl.BlockSpec((BT, TOPK), lambda i: (i, 0)),
        ],
        out_shape=[
            jax.ShapeDtypeStruct((T, TOPK), jnp.float32),
            jax.ShapeDtypeStruct((T, TOPK), jnp.int32),
        ],
        compiler_params=pltpu.CompilerParams(
            vmem_limit_bytes=128 * 1024 * 1024,
        ),
    )(xf, W, b.reshape(1, E))
    return wts.reshape(B, S, TOPK), idx.reshape(B, S, TOPK)


# speedup vs baseline: 2.4542x; 1.6300x over previous
import jax
import jax.numpy as jnp
from jax.experimental import pallas as pl
from jax.experimental.pallas import tpu as pltpu

TOPK = 8


def _router_block(x_ref, w_ref, b_ref, wts_ref, idx_ref):
    x = x_ref[...]
    w = w_ref[...]
    logits = jax.lax.dot_general(
        w.astype(jnp.bfloat16), x.astype(jnp.bfloat16),
        (((1,), (1,)), ((), ())),
        preferred_element_type=jnp.float32,
    )
    logits = logits + b_ref[...]
    e, bt = logits.shape
    eidx = jax.lax.broadcasted_iota(jnp.int32, (e, bt), 0)
    vals = logits
    vs, idxs = [], []
    for _ in range(TOPK):
        m = jnp.max(vals, axis=0, keepdims=True)
        am = jnp.min(jnp.where(vals == m, eidx, e), axis=0, keepdims=True)
        vs.append(m)
        idxs.append(am)
        vals = jnp.where(eidx == am, -jnp.inf, vals)
    v = jnp.concatenate(vs, axis=0)
    i = jnp.concatenate(idxs, axis=0)
    ex = jnp.exp(v - v[:1])
    wts_ref[...] = ex / jnp.sum(ex, axis=0, keepdims=True)
    idx_ref[...] = i


def kernel(x, W, b):
    B, S, H = x.shape
    E = W.shape[0]
    T = B * S
    BT = 1024
    xf = x.reshape(T, H)
    wts, idx = pl.pallas_call(
        _router_block,
        grid=(T // BT,),
        in_specs=[
            pl.BlockSpec((BT, H), lambda i: (i, 0)),
            pl.BlockSpec((E, H), lambda i: (0, 0)),
            pl.BlockSpec((E, 1), lambda i: (0, 0)),
        ],
        out_specs=[
            pl.BlockSpec((TOPK, BT), lambda i: (0, i)),
            pl.BlockSpec((TOPK, BT), lambda i: (0, i)),
        ],
        out_shape=[
            jax.ShapeDtypeStruct((TOPK, T), jnp.float32),
            jax.ShapeDtypeStruct((TOPK, T), jnp.int32),
        ],
        compiler_params=pltpu.CompilerParams(
            vmem_limit_bytes=128 * 1024 * 1024,
        ),
    )(xf, W, b.reshape(E, 1))
    return (wts.T.reshape(B, S, TOPK), idx.T.reshape(B, S, TOPK))
